# baseline (speedup 1.0000x reference)
import functools

import jax
import jax.numpy as jnp
from jax import lax
from jax.experimental import pallas as pl
from jax.experimental.pallas import tpu as pltpu
from jax.experimental.pallas import tpu_sc as plsc

_SATELLITE_SPACER_LEN = 17
_NUM_GSPS = 360


def _pad_table_tc(table_t, width):
    E, V = table_t.shape

    def body(t_ref, out_ref):
        shifted = t_ref[:, _NUM_GSPS:].T
        padded = jnp.concatenate(
            [shifted, jnp.zeros((V - _NUM_GSPS, width - E), jnp.float32)],
            axis=1)
        out_ref[...] = jnp.concatenate(
            [padded, jnp.zeros((_NUM_GSPS, width), jnp.float32)], axis=0)

    return pl.pallas_call(
        body,
        out_shape=jax.ShapeDtypeStruct((V, width), jnp.float32),
    )(table_t)


def _sc_embedding_gather(table, idx):
    B, N = idx.shape
    V, Ep = table.shape
    info = plsc.get_sparse_core_info()
    L = info.num_lanes
    num_workers = info.num_cores * info.num_subcores
    rows_per_w = max(1, B // num_workers)

    mesh = plsc.VectorSubcoreMesh(core_axis_name="c", subcore_axis_name="s")

    @functools.partial(
        pl.kernel,
        mesh=mesh,
        out_type=jax.ShapeDtypeStruct((B, N, Ep), jnp.float32),
        scratch_types=[
            pltpu.VMEM((N,), jnp.int32),
            pltpu.VMEM((N, Ep), jnp.float32),
            pltpu.SemaphoreType.DMA,
        ],
    )
    def gather_kernel(table_hbm, idx_hbm, out_hbm, idx_v, rows_v, sem):
        wid = lax.axis_index("s") * info.num_cores + lax.axis_index("c")

        @pl.when(wid * rows_per_w < B)
        def _():
            for r in range(rows_per_w):
                row = wid * rows_per_w + r
                pltpu.sync_copy(idx_hbm.at[row], idx_v)
                pltpu.async_copy(table_hbm.at[idx_v], rows_v, sem).wait()
                pltpu.sync_copy(rows_v, out_hbm.at[row])

    return gather_kernel(table, idx)


def _assemble(t0s, tf_t, tft0, az, el, yf_t, xf_t, emb, pv, e_dim,
              interpret=False):
    B, Ft, T = tf_t.shape
    _, Fp, N = yf_t.shape
    Ep = emb.shape[-1]
    E = e_dim
    D = Ft + Ft + 2 + Fp + Fp + _SATELLITE_SPACER_LEN + 1 + E + 1

    BB = 4 if B % 4 == 0 else 1
    Z = 2 * Ft + 2 + 2 * Fp
    DA = Z + _SATELLITE_SPACER_LEN + 1

    def body_a(tf_ref, tft0_ref, az_ref, el_ref, y_ref, x_ref, out_ref):
        for r in range(BB):
            b = pl.program_id(0) * BB + r
            rows = slice(r * T, (r + 1) * T)
            t_f = tf_ref[r].T
            t0 = tft0_ref[b]
            az_ = az_ref[b]
            el_ = el_ref[b]
            base = jnp.concatenate([
                t_f,
                jnp.broadcast_to(t0[None, :], (T, Ft)),
                az_[:, None],
                el_[:, None],
            ], axis=1)
            for d in range(2 * Ft + 2):
                out_ref[d, rows] = jnp.broadcast_to(base[:, d:d + 1], (T, N))
            for d in range(Fp):
                out_ref[2 * Ft + 2 + d, rows] = jnp.broadcast_to(
                    y_ref[r, d:d + 1, :], (T, N))
                out_ref[2 * Ft + 2 + Fp + d, rows] = jnp.broadcast_to(
                    x_ref[r, d:d + 1, :], (T, N))
            out_ref[Z:DA, rows] = jnp.zeros(
                (_SATELLITE_SPACER_LEN + 1, T, N), jnp.float32)

    part_a = pl.pallas_call(
        body_a,
        grid=(B // BB,),
        in_specs=[
            pl.BlockSpec((BB, Ft, T), lambda b: (b, 0, 0)),
            pl.BlockSpec((B, Ft), lambda b: (0, 0)),
            pl.BlockSpec((B, T), lambda b: (0, 0)),
            pl.BlockSpec((B, T), lambda b: (0, 0)),
            pl.BlockSpec((BB, Fp, N), lambda b: (b, 0, 0)),
            pl.BlockSpec((BB, Fp, N), lambda b: (b, 0, 0)),
        ],
        out_specs=pl.BlockSpec((DA, BB * T, N), lambda b: (0, b, 0)),
        out_shape=jax.ShapeDtypeStruct((D, B * T, N), jnp.float32),
        interpret=interpret,
    )(tf_t, tft0, az, el, yf_t, xf_t)

    EG = 4
    n_eg = E // EG

    def body_b(t0_ref, emb_ref, pv_ref, prev_ref, out_ref):
        del prev_ref
        e = pl.program_id(1)
        for r in range(BB):
            rows = slice(r * T, (r + 1) * T)
            for k in range(n_eg):
                @pl.when(e == k)
                def _(r=r, k=k, rows=rows):
                    e_t = emb_ref[r][:, k * EG:(k + 1) * EG].T
                    for d in range(EG):
                        out_ref[d, rows] = jnp.broadcast_to(
                            e_t[d:d + 1, :], (T, N))

            @pl.when(e == n_eg)
            def _(r=r, rows=rows):
                t_ids = lax.broadcasted_iota(jnp.int32, (T, N), 0)
                out_ref[0, rows] = jnp.where(
                    t_ids <= t0_ref[0], pv_ref[r], 0.0)
                for d in range(1, EG):
                    out_ref[d, rows] = jnp.zeros((T, N), jnp.float32)

    return pl.pallas_call(
        body_b,
        grid=(B // BB, n_eg + 1),
        in_specs=[
            pl.BlockSpec(memory_space=pltpu.SMEM),
            pl.BlockSpec((BB, N, Ep), lambda b, e: (b, 0, 0)),
            pl.BlockSpec((BB, T, N), lambda b, e: (b, 0, 0)),
            pl.BlockSpec(memory_space=pl.ANY),
        ],
        out_specs=pl.BlockSpec((EG, BB * T, N), lambda b, e: (DA // EG + e, b, 0)),
        out_shape=jax.ShapeDtypeStruct((D, B * T, N), jnp.float32),
        input_output_aliases={3: 0},
        interpret=interpret,
    )(t0s, emb, pv, part_a)


def kernel(pv, pv_solar_azimuth, pv_solar_elevation, pv_time_utc_fourier,
           pv_time_utc_fourier_t0, pv_y_osgb_fourier, pv_x_osgb_fourier,
           pv_system_row_number, pv_t0_idx, embedding_table):
    B, T, N = pv.shape
    E = embedding_table.shape[-1]
    table_p = _pad_table_tc(jnp.transpose(embedding_table, (1, 0)), 128)
    emb = _sc_embedding_gather(table_p,
                               pv_system_row_number.astype(jnp.int32))
    t0s = jnp.asarray(pv_t0_idx, jnp.int32).reshape(1)
    out = _assemble(
        t0s,
        jnp.transpose(pv_time_utc_fourier, (0, 2, 1)),
        pv_time_utc_fourier_t0,
        pv_solar_azimuth,
        pv_solar_elevation,
        jnp.transpose(pv_y_osgb_fourier, (0, 2, 1)),
        jnp.transpose(pv_x_osgb_fourier, (0, 2, 1)),
        emb,
        pv,
        e_dim=E,
    )
    return jnp.transpose(out, (1, 2, 0))


# speedup vs baseline: 1.3510x; 1.3510x over previous
import functools

import jax
import jax.numpy as jnp
from jax import lax
from jax.experimental import pallas as pl
from jax.experimental.pallas import tpu as pltpu
from jax.experimental.pallas import tpu_sc as plsc

_SATELLITE_SPACER_LEN = 17
_NUM_GSPS = 360


def _pad_table_tc(table_t, width):
    E, V = table_t.shape

    def body(t_ref, out_ref):
        shifted = t_ref[:, _NUM_GSPS:].T
        padded = jnp.concatenate(
            [shifted, jnp.zeros((V - _NUM_GSPS, width - E), jnp.float32)],
            axis=1)
        out_ref[...] = jnp.concatenate(
            [padded, jnp.zeros((_NUM_GSPS, width), jnp.float32)], axis=0)

    return pl.pallas_call(
        body,
        out_shape=jax.ShapeDtypeStruct((V, width), jnp.float32),
    )(table_t)


def _sc_embedding_gather(table, idx):
    B, N = idx.shape
    V, Ep = table.shape
    info = plsc.get_sparse_core_info()
    L = info.num_lanes
    num_workers = info.num_cores * info.num_subcores
    rows_per_w = max(1, B // num_workers)

    mesh = plsc.VectorSubcoreMesh(core_axis_name="c", subcore_axis_name="s")

    @functools.partial(
        pl.kernel,
        mesh=mesh,
        out_type=jax.ShapeDtypeStruct((B, N, Ep), jnp.float32),
        scratch_types=[
            pltpu.VMEM((N,), jnp.int32),
            pltpu.VMEM((N, Ep), jnp.float32),
            pltpu.SemaphoreType.DMA,
        ],
    )
    def gather_kernel(table_hbm, idx_hbm, out_hbm, idx_v, rows_v, sem):
        wid = lax.axis_index("s") * info.num_cores + lax.axis_index("c")

        @pl.when(wid * rows_per_w < B)
        def _():
            for r in range(rows_per_w):
                row = wid * rows_per_w + r
                pltpu.sync_copy(idx_hbm.at[row], idx_v)
                pltpu.async_copy(table_hbm.at[idx_v], rows_v, sem).wait()
                pltpu.sync_copy(rows_v, out_hbm.at[row])

    return gather_kernel(table, idx)


def _assemble(t0s, tf_t, tft0, az, el, yf_t, xf_t, emb, pv, e_dim,
              interpret=False):
    B, Ft, T = tf_t.shape
    _, Fp, N = yf_t.shape
    Ep = emb.shape[-1]
    E = e_dim
    D = Ft + Ft + 2 + Fp + Fp + _SATELLITE_SPACER_LEN + 1 + E + 1

    BB = 4 if B % 4 == 0 else 1
    Z = 2 * Ft + 2 + 2 * Fp
    DA = Z + _SATELLITE_SPACER_LEN + 1

    def body_a(tf_ref, tft0_ref, az_ref, el_ref, y_ref, x_ref, out_ref):
        for r in range(BB):
            b = pl.program_id(0) * BB + r
            rows = slice(r * T, (r + 1) * T)
            t_f = tf_ref[r].T
            t0 = tft0_ref[b]
            az_ = az_ref[b]
            el_ = el_ref[b]
            base = jnp.concatenate([
                t_f,
                jnp.broadcast_to(t0[None, :], (T, Ft)),
                az_[:, None],
                el_[:, None],
            ], axis=1)
            for d in range(2 * Ft + 2):
                out_ref[d, rows] = jnp.broadcast_to(base[:, d:d + 1], (T, N))
            for d in range(Fp):
                out_ref[2 * Ft + 2 + d, rows] = jnp.broadcast_to(
                    y_ref[r, d:d + 1, :], (T, N))
                out_ref[2 * Ft + 2 + Fp + d, rows] = jnp.broadcast_to(
                    x_ref[r, d:d + 1, :], (T, N))
            out_ref[Z:DA, rows] = jnp.zeros(
                (_SATELLITE_SPACER_LEN + 1, T, N), jnp.float32)

    part_a = pl.pallas_call(
        body_a,
        grid=(B // BB,),
        in_specs=[
            pl.BlockSpec((BB, Ft, T), lambda b: (b, 0, 0)),
            pl.BlockSpec((B, Ft), lambda b: (0, 0)),
            pl.BlockSpec((B, T), lambda b: (0, 0)),
            pl.BlockSpec((B, T), lambda b: (0, 0)),
            pl.BlockSpec((BB, Fp, N), lambda b: (b, 0, 0)),
            pl.BlockSpec((BB, Fp, N), lambda b: (b, 0, 0)),
        ],
        out_specs=pl.BlockSpec((DA, BB * T, N), lambda b: (0, b, 0)),
        out_shape=jax.ShapeDtypeStruct((D, B * T, N), jnp.float32),
        interpret=interpret,
    )(tf_t, tft0, az, el, yf_t, xf_t)

    EG = 4
    n_eg = E // EG

    def body_b(t0_ref, emb_ref, pv_ref, prev_ref, out_ref):
        del prev_ref
        e = pl.program_id(0)
        for k in range(n_eg):
            @pl.when(e == k)
            def _(k=k):
                for b in range(B):
                    e_t = emb_ref[b][:, k * EG:(k + 1) * EG].T
                    for d in range(EG):
                        out_ref[d, b * T:(b + 1) * T] = jnp.broadcast_to(
                            e_t[d:d + 1, :], (T, N))

        @pl.when(e == n_eg)
        def _():
            t_ids = lax.broadcasted_iota(jnp.int32, (T, N), 0)
            for b in range(B):
                out_ref[0, b * T:(b + 1) * T] = jnp.where(
                    t_ids <= t0_ref[0], pv_ref[b], 0.0)
            for d in range(1, EG):
                out_ref[d] = jnp.zeros((B * T, N), jnp.float32)

    return pl.pallas_call(
        body_b,
        grid=(n_eg + 1,),
        in_specs=[
            pl.BlockSpec(memory_space=pltpu.SMEM),
            pl.BlockSpec((B, N, Ep), lambda e: (0, 0, 0)),
            pl.BlockSpec((B, T, N), lambda e: (0, 0, 0)),
            pl.BlockSpec(memory_space=pl.ANY),
        ],
        out_specs=pl.BlockSpec((EG, B * T, N), lambda e: (DA // EG + e, 0, 0)),
        out_shape=jax.ShapeDtypeStruct((D, B * T, N), jnp.float32),
        input_output_aliases={3: 0},
        interpret=interpret,
    )(t0s, emb, pv, part_a)


def kernel(pv, pv_solar_azimuth, pv_solar_elevation, pv_time_utc_fourier,
           pv_time_utc_fourier_t0, pv_y_osgb_fourier, pv_x_osgb_fourier,
           pv_system_row_number, pv_t0_idx, embedding_table):
    B, T, N = pv.shape
    E = embedding_table.shape[-1]
    table_p = _pad_table_tc(jnp.transpose(embedding_table, (1, 0)), 128)
    emb = _sc_embedding_gather(table_p,
                               pv_system_row_number.astype(jnp.int32))
    t0s = jnp.asarray(pv_t0_idx, jnp.int32).reshape(1)
    out = _assemble(
        t0s,
        jnp.transpose(pv_time_utc_fourier, (0, 2, 1)),
        pv_time_utc_fourier_t0,
        pv_solar_azimuth,
        pv_solar_elevation,
        jnp.transpose(pv_y_osgb_fourier, (0, 2, 1)),
        jnp.transpose(pv_x_osgb_fourier, (0, 2, 1)),
        emb,
        pv,
        e_dim=E,
    )
    return jnp.transpose(out, (1, 2, 0))


# speedup vs baseline: 1.4301x; 1.0585x over previous
import functools

import jax
import jax.numpy as jnp
from jax import lax
from jax.experimental import pallas as pl
from jax.experimental.pallas import tpu as pltpu
from jax.experimental.pallas import tpu_sc as plsc

_SATELLITE_SPACER_LEN = 17
_NUM_GSPS = 360


def _pad_table_tc(table_t, width):
    E, V = table_t.shape

    def body(t_ref, out_ref):
        shifted = t_ref[:, _NUM_GSPS:].T
        padded = jnp.concatenate(
            [shifted, jnp.zeros((V - _NUM_GSPS, width - E), jnp.float32)],
            axis=1)
        out_ref[...] = jnp.concatenate(
            [padded, jnp.zeros((_NUM_GSPS, width), jnp.float32)], axis=0)

    return pl.pallas_call(
        body,
        out_shape=jax.ShapeDtypeStruct((V, width), jnp.float32),
    )(table_t)


def _sc_embedding_gather(table, idx):
    B, N = idx.shape
    V, Ep = table.shape
    info = plsc.get_sparse_core_info()
    L = info.num_lanes
    num_workers = info.num_cores * info.num_subcores
    rows_per_w = max(1, B // num_workers)

    mesh = plsc.VectorSubcoreMesh(core_axis_name="c", subcore_axis_name="s")

    @functools.partial(
        pl.kernel,
        mesh=mesh,
        out_type=jax.ShapeDtypeStruct((B, N, Ep), jnp.float32),
        scratch_types=[
            pltpu.VMEM((N,), jnp.int32),
            pltpu.VMEM((N, Ep), jnp.float32),
            pltpu.SemaphoreType.DMA,
        ],
    )
    def gather_kernel(table_hbm, idx_hbm, out_hbm, idx_v, rows_v, sem):
        wid = lax.axis_index("s") * info.num_cores + lax.axis_index("c")

        @pl.when(wid * rows_per_w < B)
        def _():
            for r in range(rows_per_w):
                row = wid * rows_per_w + r
                pltpu.sync_copy(idx_hbm.at[row], idx_v)
                pltpu.async_copy(table_hbm.at[idx_v], rows_v, sem).wait()
                pltpu.sync_copy(rows_v, out_hbm.at[row])

    return gather_kernel(table, idx)


def _assemble(t0s, tf_t, tft0, az, el, yf_t, xf_t, emb, pv, e_dim,
              interpret=False):
    B, Ft, T = tf_t.shape
    _, Fp, N = yf_t.shape
    Ep = emb.shape[-1]
    E = e_dim
    D = Ft + Ft + 2 + Fp + Fp + _SATELLITE_SPACER_LEN + 1 + E + 1

    BB = 4 if B % 4 == 0 else 1

    def body(t0_ref, tf_ref, tft0_ref, az_ref, el_ref, y_ref, x_ref,
             emb_ref, pv_ref, out_ref):
        for r in range(BB):
            b = pl.program_id(0) * BB + r
            rows = slice(r * T, (r + 1) * T)
            t_f = tf_ref[r].T
            t0 = tft0_ref[b]
            az_ = az_ref[b]
            el_ = el_ref[b]
            base = jnp.concatenate([
                t_f,
                jnp.broadcast_to(t0[None, :], (T, Ft)),
                az_[:, None],
                el_[:, None],
            ], axis=1)
            for d in range(2 * Ft + 2):
                out_ref[d, rows] = jnp.broadcast_to(base[:, d:d + 1], (T, N))
            for d in range(Fp):
                out_ref[2 * Ft + 2 + d, rows] = jnp.broadcast_to(
                    y_ref[r, d:d + 1, :], (T, N))
                out_ref[2 * Ft + 2 + Fp + d, rows] = jnp.broadcast_to(
                    x_ref[r, d:d + 1, :], (T, N))
            e_t = emb_ref[r][:, :E].T
            for d in range(E):
                out_ref[D - 1 - E + d, rows] = jnp.broadcast_to(
                    e_t[d:d + 1, :], (T, N))
            z0 = 2 * Ft + 2 + 2 * Fp
            out_ref[z0:z0 + _SATELLITE_SPACER_LEN + 1, rows] = jnp.zeros(
                (_SATELLITE_SPACER_LEN + 1, T, N), jnp.float32)
            t_ids = lax.broadcasted_iota(jnp.int32, (T, N), 0)
            out_ref[D - 1, rows] = jnp.where(
                t_ids <= t0_ref[0], pv_ref[r], 0.0)

    return pl.pallas_call(
        body,
        grid=(B // BB,),
        in_specs=[
            pl.BlockSpec(memory_space=pltpu.SMEM),
            pl.BlockSpec((BB, Ft, T), lambda b: (b, 0, 0)),
            pl.BlockSpec((B, Ft), lambda b: (0, 0)),
            pl.BlockSpec((B, T), lambda b: (0, 0)),
            pl.BlockSpec((B, T), lambda b: (0, 0)),
            pl.BlockSpec((BB, Fp, N), lambda b: (b, 0, 0)),
            pl.BlockSpec((BB, Fp, N), lambda b: (b, 0, 0)),
            pl.BlockSpec((BB, N, Ep), lambda b: (b, 0, 0)),
            pl.BlockSpec((BB, T, N), lambda b: (b, 0, 0)),
        ],
        out_specs=pl.BlockSpec((D, BB * T, N), lambda b: (0, b, 0)),
        out_shape=jax.ShapeDtypeStruct((D, B * T, N), jnp.float32),
        interpret=interpret,
    )(t0s, tf_t, tft0, az, el, yf_t, xf_t, emb, pv)


def kernel(pv, pv_solar_azimuth, pv_solar_elevation, pv_time_utc_fourier,
           pv_time_utc_fourier_t0, pv_y_osgb_fourier, pv_x_osgb_fourier,
           pv_system_row_number, pv_t0_idx, embedding_table):
    B, T, N = pv.shape
    E = embedding_table.shape[-1]
    table_p = _pad_table_tc(jnp.transpose(embedding_table, (1, 0)), 128)
    emb = _sc_embedding_gather(table_p,
                               pv_system_row_number.astype(jnp.int32))
    t0s = jnp.asarray(pv_t0_idx, jnp.int32).reshape(1)
    out = _assemble(
        t0s,
        jnp.transpose(pv_time_utc_fourier, (0, 2, 1)),
        pv_time_utc_fourier_t0,
        pv_solar_azimuth,
        pv_solar_elevation,
        jnp.transpose(pv_y_osgb_fourier, (0, 2, 1)),
        jnp.transpose(pv_x_osgb_fourier, (0, 2, 1)),
        emb,
        pv,
        e_dim=E,
    )
    return jnp.transpose(out, (1, 2, 0))
